# baseline (speedup 1.0000x reference)
import functools

import jax
import jax.numpy as jnp
from jax import lax
from jax.experimental import pallas as pl
from jax.experimental.pallas import tpu as pltpu
from jax.experimental.pallas import tpu_sc as plsc


_K = 512
_D = 128
_RB = 128
_N_FLAT = 8 * 28 * 28


def _vq_body(flat_ref, cb_ref, idx_ref, oh_ref):
    flat = flat_ref[...]
    cb = cb_ref[...]
    rn = jnp.sum(flat ** 2, axis=1, keepdims=True)
    cn = jnp.sum(cb ** 2, axis=1)
    prod = lax.dot_general(flat, cb, (((1,), (1,)), ((), ())),
                           preferred_element_type=jnp.float32)
    d = rn - 2.0 * prod + cn[None, :]
    dmin = jnp.min(d, axis=1, keepdims=True)
    iota = lax.broadcasted_iota(jnp.int32, d.shape, 1)
    idxv = jnp.min(jnp.where(d == dmin, iota, _K), axis=1)
    oh_ref[...] = (iota == idxv[:, None]).astype(jnp.float32)
    idx_ref[...] = idxv.reshape(1, 1, _RB)


def _vq_tc(flat, codebook):
    nblk = _N_FLAT // _RB
    idx3, onehot = pl.pallas_call(
        _vq_body,
        grid=(nblk,),
        in_specs=[
            pl.BlockSpec((_RB, _D), lambda i: (i, 0)),
            pl.BlockSpec((_K, _D), lambda i: (0, 0)),
        ],
        out_specs=[
            pl.BlockSpec((1, 1, _RB), lambda i: (i, 0, 0)),
            pl.BlockSpec((_RB, _K), lambda i: (i, 0)),
        ],
        out_shape=[
            jax.ShapeDtypeStruct((nblk, 1, _RB), jnp.int32),
            jax.ShapeDtypeStruct((_N_FLAT, _K), jnp.float32),
        ],
    )(flat, codebook)
    return idx3.reshape(_N_FLAT), onehot


_NW = 32
_BPAD = 6400
_BPW = _BPAD // _NW


def _sc_gather(codebook, idx_pad):
    mesh = plsc.VectorSubcoreMesh(core_axis_name="c", subcore_axis_name="s")

    @functools.partial(
        pl.kernel, mesh=mesh,
        out_type=jax.ShapeDtypeStruct((_BPAD, _D), jnp.float32),
        scratch_types=[
            pltpu.VMEM((_BPW,), jnp.int32),
            pltpu.VMEM((_BPW, _D), jnp.float32),
            pltpu.SemaphoreType.DMA,
        ],
    )
    def k(table_hbm, idx_hbm, out_hbm, idx_v, rows_v, sem):
        wid = lax.axis_index("s") * 2 + lax.axis_index("c")
        base = wid * _BPW
        pltpu.sync_copy(idx_hbm.at[pl.ds(base, _BPW)], idx_v)
        pltpu.async_copy(table_hbm.at[idx_v], rows_v, sem).wait()
        pltpu.sync_copy(rows_v, out_hbm.at[pl.ds(base, _BPW)])

    return k(codebook, idx_pad)


def _collapse_w(w4):
    re = jnp.stack([w4[0], w4[1] + w4[2], w4[3]], axis=0)
    ro = jnp.stack([w4[0] + w4[1], w4[2] + w4[3]], axis=0)

    def colc(wr):
        ce = jnp.stack([wr[:, 0], wr[:, 1] + wr[:, 2], wr[:, 3]], axis=1)
        co = jnp.stack([wr[:, 0] + wr[:, 1], wr[:, 2] + wr[:, 3]], axis=1)
        return ce, co

    wee, weo = colc(re)
    woe, woo = colc(ro)

    def m(w):
        t, s, ci, co = w.shape
        if s == 3:
            w = w[:, jnp.array([1, 2, 0])]
        return w.reshape(t, s * ci, co)

    return m(wee), m(weo), m(woe), m(woo)


def _dec_stage_body(Hc, W, Ci, Co, apply_act, rc,
                    x_ref, a_ref, c_ref, wee_ref, weo_ref, woe_ref, woo_ref,
                    b_ref, ue_ref, uo_ref, st_ref):
    xr = x_ref[0, 0]
    if apply_act:
        xr = jnp.maximum(xr * a_ref[0] + c_ref[0], 0.0)
        k = pl.program_id(1)
        r = lax.broadcasted_iota(jnp.int32, (Hc + 2, 1, 1), 0)
        valid = ((r != 0) | (k != 0)) & ((r != Hc + 1) | (k != rc - 1))
        xr = jnp.where(valid, xr, 0.0)
    zc = jnp.zeros((Hc + 2, 1, Ci), jnp.float32)
    xp = jnp.concatenate([zc, xr, zc], axis=1)
    c3 = jnp.concatenate([xp[:, 1:W + 1], xp[:, 2:W + 2], xp[:, 0:W]],
                         axis=-1)
    c2 = c3[:, :, :2 * Ci]

    bias = b_ref[0]

    def mm(slab, wmat):
        kk = slab.shape[-1]
        return lax.dot_general(slab.reshape(Hc * W, kk), wmat,
                               (((1,), (0,)), ((), ())),
                               precision=lax.Precision.HIGHEST,
                               preferred_element_type=jnp.float32)

    pee = mm(c3[0:Hc], wee_ref[0]) + mm(c3[1:Hc + 1], wee_ref[1]) \
        + mm(c3[2:Hc + 2], wee_ref[2]) + bias
    peo = mm(c2[0:Hc], weo_ref[0]) + mm(c2[1:Hc + 1], weo_ref[1]) \
        + mm(c2[2:Hc + 2], weo_ref[2]) + bias
    poe = mm(c3[1:Hc + 1], woe_ref[0]) + mm(c3[2:Hc + 2], woe_ref[1]) + bias
    poo = mm(c2[1:Hc + 1], woo_ref[0]) + mm(c2[2:Hc + 2], woo_ref[1]) + bias

    ue_ref[0, 0] = jnp.concatenate(
        [pee.reshape(Hc, W, Co), peo.reshape(Hc, W, Co)], axis=-1)
    uo_ref[0, 0] = jnp.concatenate(
        [poe.reshape(Hc, W, Co), poo.reshape(Hc, W, Co)], axis=-1)
    s = (jnp.sum(pee, axis=0) + jnp.sum(peo, axis=0)
         + jnp.sum(poe, axis=0) + jnp.sum(poo, axis=0))
    ss = (jnp.sum(pee * pee, axis=0) + jnp.sum(peo * peo, axis=0)
          + jnp.sum(poe * poe, axis=0) + jnp.sum(poo * poo, axis=0))
    st_ref[0, 0, 0] = s
    st_ref[0, 0, 1] = ss


def _dec_stage(x, a, c, w4, bias, apply_act, rc=1):
    B, H, W, Ci = x.shape
    Co = w4.shape[-1]
    Hc = H // rc
    wee, weo, woe, woo = _collapse_w(w4)
    xpad = jnp.pad(x, ((0, 0), (1, 1), (0, 0), (0, 0)))
    if rc > 1:
        win = jnp.stack([xpad[:, k * Hc:k * Hc + Hc + 2] for k in range(rc)],
                        axis=1)
    else:
        win = xpad[:, None]
    body = functools.partial(_dec_stage_body, Hc, W, Ci, Co, apply_act, rc)
    full = lambda s: pl.BlockSpec(s, lambda b, k: tuple(0 for _ in s))
    outs = pl.pallas_call(
        body,
        grid=(B, rc),
        in_specs=[
            pl.BlockSpec((1, 1, Hc + 2, W, Ci), lambda b, k: (b, k, 0, 0, 0)),
            full((1, Ci)), full((1, Ci)),
            full(wee.shape), full(weo.shape), full(woe.shape),
            full(woo.shape), full((1, Co)),
        ],
        out_specs=[
            pl.BlockSpec((1, 1, Hc, W, 2 * Co), lambda b, k: (b, k, 0, 0, 0)),
            pl.BlockSpec((1, 1, Hc, W, 2 * Co), lambda b, k: (b, k, 0, 0, 0)),
            pl.BlockSpec((1, 1, 2, Co), lambda b, k: (b, k, 0, 0)),
        ],
        out_shape=[
            jax.ShapeDtypeStruct((B, rc, Hc, W, 2 * Co), jnp.float32),
            jax.ShapeDtypeStruct((B, rc, Hc, W, 2 * Co), jnp.float32),
            jax.ShapeDtypeStruct((B, rc, 2, Co), jnp.float32),
        ],
    )(win, a.reshape(1, Ci), c.reshape(1, Ci), wee, weo, woe, woo,
      bias.reshape(1, Co))
    ue, uo, st = outs
    ue = ue.reshape(B, H, W, 2 * Co)
    uo = uo.reshape(B, H, W, 2 * Co)
    o = jnp.stack([ue, uo], axis=2)
    return o.reshape(B, 2 * H, 2 * W, Co), jnp.sum(st, axis=(0, 1))


def _bn_coeffs(stats, n, gamma, beta, eps=1e-5):
    m = stats[0] / n
    v = stats[1] / n - m * m
    a = gamma / jnp.sqrt(v + eps)
    return a, beta - m * a


def _out_body(x_ref, a_ref, c_ref, w5_ref, bo_ref, rec_ref):
    HO, WO = 55, 220
    x = jnp.maximum(x_ref[0, 0] * a_ref[0] + c_ref[0], 0.0)
    r5 = jnp.concatenate([x[u:u + HO] for u in range(5)], axis=-1)
    g5 = lax.dot_general(r5.reshape(HO * 224, 160), w5_ref[...],
                         (((1,), (0,)), ((), ())),
                         precision=lax.Precision.HIGHEST,
                         preferred_element_type=jnp.float32)
    g5 = g5.reshape(HO, 224, 5)
    acc = g5[:, 0:WO, 0]
    for v in range(1, 5):
        acc = acc + g5[:, v:v + WO, v]
    rec_ref[0, 0] = jnp.maximum(acc + bo_ref[0, 0], 0.0)


def _dec_out(x, a, c, wo, bo):
    B = x.shape[0]
    w5 = jnp.transpose(wo[:, :, :, 0], (0, 2, 1)).reshape(160, 5)
    win = jnp.stack([x[:, 55 * k:55 * k + 59] for k in range(4)], axis=1)
    rec = pl.pallas_call(
        _out_body,
        grid=(B, 4),
        in_specs=[
            pl.BlockSpec((1, 1, 59, 224, 32), lambda b, k: (b, k, 0, 0, 0)),
            pl.BlockSpec((1, 32), lambda b, k: (0, 0)),
            pl.BlockSpec((1, 32), lambda b, k: (0, 0)),
            pl.BlockSpec((160, 5), lambda b, k: (0, 0)),
            pl.BlockSpec((1, 1), lambda b, k: (0, 0)),
        ],
        out_specs=pl.BlockSpec((1, 1, 55, 220), lambda b, k: (b, k, 0, 0)),
        out_shape=jax.ShapeDtypeStruct((B, 4, 55, 220), jnp.float32),
    )(win, a.reshape(1, 32), c.reshape(1, 32), w5, bo.reshape(1, 1))
    return rec.reshape(B, 220, 220, 1)


def _conv(x, w, b, stride=1, padding='SAME'):
    y = lax.conv_general_dilated(x, w, (stride, stride), padding,
                                 dimension_numbers=('NHWC', 'HWIO', 'NHWC'))
    return y + b


def _bn(x, gamma, beta, eps=1e-5):
    m = jnp.mean(x, axis=(0, 1, 2), keepdims=True)
    v = jnp.var(x, axis=(0, 1, 2), keepdims=True)
    return gamma * (x - m) / jnp.sqrt(v + eps) + beta


def _upsample(x):
    return jnp.repeat(jnp.repeat(x, 2, axis=1), 2, axis=2)


def kernel(img, We1, be1, We2, be2, We3, be3, codebook,
           Wd1, bd1, g1, bb1, Wd2, bd2, g2, bb2, Wd3, bd3, g3, bb3, Wo, bo):
    x = _conv(img, We1, be1, 2)
    x = _conv(x, We2, be2, 2)
    encoded = _conv(x, We3, be3, 2)

    flat = encoded.reshape((-1, _D))
    idx, discrete = _vq_tc(flat, codebook)

    idx_pad = jnp.concatenate(
        [idx, jnp.zeros((_BPAD - _N_FLAT,), jnp.int32)])
    qflat = _sc_gather(codebook, idx_pad)[:_N_FLAT]
    quantized = qflat.reshape(encoded.shape)

    zero128 = jnp.zeros((128,), jnp.float32)
    one128 = jnp.ones((128,), jnp.float32)
    o1, st1 = _dec_stage(quantized, one128, zero128, Wd1, bd1, False)
    a1, c1 = _bn_coeffs(st1, 8 * 56 * 56, g1, bb1)
    o2, st2 = _dec_stage(o1, a1, c1, Wd2, bd2, True)
    a2, c2 = _bn_coeffs(st2, 8 * 112 * 112, g2, bb2)
    o3, st3 = _dec_stage(o2, a2, c2, Wd3, bd3, True, rc=4)
    a3, c3 = _bn_coeffs(st3, 8 * 224 * 224, g3, bb3)
    rec = _dec_out(o3, a3, c3, Wo, bo)
    return (rec, encoded, discrete, quantized)


# speedup vs baseline: 1.3488x; 1.3488x over previous
import functools

import jax
import jax.numpy as jnp
from jax import lax
from jax.experimental import pallas as pl
from jax.experimental.pallas import tpu as pltpu
from jax.experimental.pallas import tpu_sc as plsc


_K = 512
_D = 128
_RB = 128
_N_FLAT = 8 * 28 * 28


def _vq_body(flat_ref, cb_ref, idx_ref, oh_ref):
    flat = flat_ref[...]
    cb = cb_ref[...]
    rn = jnp.sum(flat ** 2, axis=1, keepdims=True)
    cn = jnp.sum(cb ** 2, axis=1)
    prod = lax.dot_general(flat, cb, (((1,), (1,)), ((), ())),
                           preferred_element_type=jnp.float32)
    d = rn - 2.0 * prod + cn[None, :]
    dmin = jnp.min(d, axis=1, keepdims=True)
    iota = lax.broadcasted_iota(jnp.int32, d.shape, 1)
    idxv = jnp.min(jnp.where(d == dmin, iota, _K), axis=1)
    oh_ref[...] = (iota == idxv[:, None]).astype(jnp.float32)
    idx_ref[...] = idxv.reshape(1, 1, _RB)


def _vq_tc(flat, codebook):
    nblk = _N_FLAT // _RB
    idx3, onehot = pl.pallas_call(
        _vq_body,
        grid=(nblk,),
        in_specs=[
            pl.BlockSpec((_RB, _D), lambda i: (i, 0)),
            pl.BlockSpec((_K, _D), lambda i: (0, 0)),
        ],
        out_specs=[
            pl.BlockSpec((1, 1, _RB), lambda i: (i, 0, 0)),
            pl.BlockSpec((_RB, _K), lambda i: (i, 0)),
        ],
        out_shape=[
            jax.ShapeDtypeStruct((nblk, 1, _RB), jnp.int32),
            jax.ShapeDtypeStruct((_N_FLAT, _K), jnp.float32),
        ],
    )(flat, codebook)
    return idx3.reshape(_N_FLAT), onehot


_NW = 32
_BPAD = 6400
_BPW = _BPAD // _NW


def _sc_gather(codebook, idx_pad):
    mesh = plsc.VectorSubcoreMesh(core_axis_name="c", subcore_axis_name="s")

    @functools.partial(
        pl.kernel, mesh=mesh,
        out_type=jax.ShapeDtypeStruct((_BPAD, _D), jnp.float32),
        scratch_types=[
            pltpu.VMEM((_BPW,), jnp.int32),
            pltpu.VMEM((_BPW, _D), jnp.float32),
            pltpu.SemaphoreType.DMA,
        ],
    )
    def k(table_hbm, idx_hbm, out_hbm, idx_v, rows_v, sem):
        wid = lax.axis_index("s") * 2 + lax.axis_index("c")
        base = wid * _BPW
        pltpu.sync_copy(idx_hbm.at[pl.ds(base, _BPW)], idx_v)
        pltpu.async_copy(table_hbm.at[idx_v], rows_v, sem).wait()
        pltpu.sync_copy(rows_v, out_hbm.at[pl.ds(base, _BPW)])

    return k(codebook, idx_pad)


def _collapse_w(w4):
    re = jnp.stack([w4[0], w4[1] + w4[2], w4[3]], axis=0)
    ro = jnp.stack([w4[0] + w4[1], w4[2] + w4[3]], axis=0)

    def colc(wr):
        ce = jnp.stack([wr[:, 0], wr[:, 1] + wr[:, 2], wr[:, 3]], axis=1)
        co = jnp.stack([wr[:, 0] + wr[:, 1], wr[:, 2] + wr[:, 3]], axis=1)
        return ce, co

    wee, weo = colc(re)
    woe, woo = colc(ro)
    ci, co = w4.shape[2], w4.shape[3]

    def m(w):
        t, s = w.shape[0], w.shape[1]
        if s == 3:
            w = w[:, jnp.array([1, 2, 0])]
        w = w.reshape(t, s * ci, co)
        if s == 2:
            w = jnp.concatenate([w, jnp.zeros((t, ci, co), jnp.float32)], 1)
        return w

    wee, weo, woe, woo = m(wee), m(weo), m(woe), m(woo)
    z1 = jnp.zeros((1, 3 * ci, co), jnp.float32)
    woe = jnp.concatenate([z1, woe], axis=0)
    woo = jnp.concatenate([z1, woo], axis=0)
    return jnp.concatenate([wee, weo, woe, woo], axis=2)


def _dec_stage_body(Hc, W, Ci, Co, apply_act, rc,
                    x_ref, a_ref, c_ref, wcat_ref, b_ref,
                    ue_ref, uo_ref, st_ref):
    xr = x_ref[0, 0]
    if apply_act:
        xr = jnp.maximum(xr * a_ref[0] + c_ref[0], 0.0)
        k = pl.program_id(1)
        r = lax.broadcasted_iota(jnp.int32, (Hc + 2, 1, 1), 0)
        valid = ((r != 0) | (k != 0)) & ((r != Hc + 1) | (k != rc - 1))
        xr = jnp.where(valid, xr, 0.0)
    zc = jnp.zeros((Hc + 2, 1, Ci), jnp.float32)
    xp = jnp.concatenate([zc, xr, zc], axis=1)
    c3 = jnp.concatenate([xp[:, 1:W + 1], xp[:, 2:W + 2], xp[:, 0:W]],
                         axis=-1)

    def mm(slab, wmat):
        return lax.dot_general(slab.reshape(Hc * W, 3 * Ci), wmat,
                               (((1,), (0,)), ((), ())),
                               precision=lax.Precision.HIGHEST,
                               preferred_element_type=jnp.float32)

    p = mm(c3[0:Hc], wcat_ref[0]) + mm(c3[1:Hc + 1], wcat_ref[1]) \
        + mm(c3[2:Hc + 2], wcat_ref[2]) + b_ref[0]
    ue_ref[0, 0] = p[:, :2 * Co].reshape(Hc, W, 2 * Co)
    uo_ref[0, 0] = p[:, 2 * Co:].reshape(Hc, W, 2 * Co)
    st_ref[0, 0, 0] = jnp.sum(p, axis=0)
    st_ref[0, 0, 1] = jnp.sum(p * p, axis=0)


def _dec_stage(x, a, c, w4, bias, apply_act, rc=1):
    B, H, W, Ci = x.shape
    Co = w4.shape[-1]
    Hc = H // rc
    wcat = _collapse_w(w4)
    bias4 = jnp.tile(bias, 4).reshape(1, 4 * Co)
    xpad = jnp.pad(x, ((0, 0), (1, 1), (0, 0), (0, 0)))
    if rc > 1:
        win = jnp.stack([xpad[:, k * Hc:k * Hc + Hc + 2] for k in range(rc)],
                        axis=1)
    else:
        win = xpad[:, None]
    body = functools.partial(_dec_stage_body, Hc, W, Ci, Co, apply_act, rc)
    full = lambda s: pl.BlockSpec(s, lambda b, k: tuple(0 for _ in s))
    outs = pl.pallas_call(
        body,
        grid=(B, rc),
        in_specs=[
            pl.BlockSpec((1, 1, Hc + 2, W, Ci), lambda b, k: (b, k, 0, 0, 0)),
            full((1, Ci)), full((1, Ci)),
            full((3, 3 * Ci, 4 * Co)), full((1, 4 * Co)),
        ],
        out_specs=[
            pl.BlockSpec((1, 1, Hc, W, 2 * Co), lambda b, k: (b, k, 0, 0, 0)),
            pl.BlockSpec((1, 1, Hc, W, 2 * Co), lambda b, k: (b, k, 0, 0, 0)),
            pl.BlockSpec((1, 1, 2, 4 * Co), lambda b, k: (b, k, 0, 0)),
        ],
        out_shape=[
            jax.ShapeDtypeStruct((B, rc, Hc, W, 2 * Co), jnp.float32),
            jax.ShapeDtypeStruct((B, rc, Hc, W, 2 * Co), jnp.float32),
            jax.ShapeDtypeStruct((B, rc, 2, 4 * Co), jnp.float32),
        ],
    )(win, a.reshape(1, Ci), c.reshape(1, Ci), wcat, bias4)
    ue, uo, st = outs
    ue = ue.reshape(B, H, W, 2 * Co)
    uo = uo.reshape(B, H, W, 2 * Co)
    o = jnp.stack([ue, uo], axis=2)
    st = jnp.sum(st, axis=(0, 1)).reshape(2, 4, Co).sum(axis=1)
    return o.reshape(B, 2 * H, 2 * W, Co), st


def _bn_coeffs(stats, n, gamma, beta, eps=1e-5):
    m = stats[0] / n
    v = stats[1] / n - m * m
    a = gamma / jnp.sqrt(v + eps)
    return a, beta - m * a


def _out_body(x_ref, a_ref, c_ref, w5_ref, bo_ref, rec_ref):
    HO, WO = 55, 220
    x = jnp.maximum(x_ref[0, 0] * a_ref[0] + c_ref[0], 0.0)
    r5 = jnp.concatenate([x[u:u + HO] for u in range(5)], axis=-1)
    g5 = lax.dot_general(r5.reshape(HO * 224, 160), w5_ref[...],
                         (((1,), (0,)), ((), ())),
                         precision=lax.Precision.HIGHEST,
                         preferred_element_type=jnp.float32)
    g5 = g5.reshape(HO, 224, 8)
    li = lax.broadcasted_iota(jnp.int32, (1, 1, 8), 2)
    acc5 = jnp.zeros((HO, WO, 8), jnp.float32)
    for v in range(5):
        acc5 = acc5 + jnp.where(li == v, g5[:, v:v + WO, :], 0.0)
    rec_ref[0, 0] = jnp.maximum(jnp.sum(acc5, axis=2) + bo_ref[0, 0], 0.0)


def _dec_out(x, a, c, wo, bo):
    B = x.shape[0]
    w5 = jnp.concatenate(
        [jnp.transpose(wo[:, :, :, 0], (0, 2, 1)).reshape(160, 5),
         jnp.zeros((160, 3), jnp.float32)], axis=1)
    win = jnp.stack([x[:, 55 * k:55 * k + 59] for k in range(4)], axis=1)
    rec = pl.pallas_call(
        _out_body,
        grid=(B, 4),
        in_specs=[
            pl.BlockSpec((1, 1, 59, 224, 32), lambda b, k: (b, k, 0, 0, 0)),
            pl.BlockSpec((1, 32), lambda b, k: (0, 0)),
            pl.BlockSpec((1, 32), lambda b, k: (0, 0)),
            pl.BlockSpec((160, 8), lambda b, k: (0, 0)),
            pl.BlockSpec((1, 1), lambda b, k: (0, 0)),
        ],
        out_specs=pl.BlockSpec((1, 1, 55, 220), lambda b, k: (b, k, 0, 0)),
        out_shape=jax.ShapeDtypeStruct((B, 4, 55, 220), jnp.float32),
    )(win, a.reshape(1, 32), c.reshape(1, 32), w5, bo.reshape(1, 1))
    return rec.reshape(B, 220, 220, 1)


def _conv(x, w, b, stride=1, padding='SAME'):
    y = lax.conv_general_dilated(x, w, (stride, stride), padding,
                                 dimension_numbers=('NHWC', 'HWIO', 'NHWC'))
    return y + b


def _bn(x, gamma, beta, eps=1e-5):
    m = jnp.mean(x, axis=(0, 1, 2), keepdims=True)
    v = jnp.var(x, axis=(0, 1, 2), keepdims=True)
    return gamma * (x - m) / jnp.sqrt(v + eps) + beta


def _upsample(x):
    return jnp.repeat(jnp.repeat(x, 2, axis=1), 2, axis=2)


def kernel(img, We1, be1, We2, be2, We3, be3, codebook,
           Wd1, bd1, g1, bb1, Wd2, bd2, g2, bb2, Wd3, bd3, g3, bb3, Wo, bo):
    x = _conv(img, We1, be1, 2)
    x = _conv(x, We2, be2, 2)
    encoded = _conv(x, We3, be3, 2)

    flat = encoded.reshape((-1, _D))
    idx, discrete = _vq_tc(flat, codebook)

    idx_pad = jnp.concatenate(
        [idx, jnp.zeros((_BPAD - _N_FLAT,), jnp.int32)])
    qflat = _sc_gather(codebook, idx_pad)[:_N_FLAT]
    quantized = qflat.reshape(encoded.shape)

    zero128 = jnp.zeros((128,), jnp.float32)
    one128 = jnp.ones((128,), jnp.float32)
    o1, st1 = _dec_stage(quantized, one128, zero128, Wd1, bd1, False)
    a1, c1 = _bn_coeffs(st1, 8 * 56 * 56, g1, bb1)
    o2, st2 = _dec_stage(o1, a1, c1, Wd2, bd2, True)
    a2, c2 = _bn_coeffs(st2, 8 * 112 * 112, g2, bb2)
    o3, st3 = _dec_stage(o2, a2, c2, Wd3, bd3, True, rc=4)
    a3, c3 = _bn_coeffs(st3, 8 * 224 * 224, g3, bb3)
    rec = _dec_out(o3, a3, c3, Wo, bo)
    return (rec, encoded, discrete, quantized)


# speedup vs baseline: 1.6613x; 1.2317x over previous
import functools

import jax
import jax.numpy as jnp
from jax import lax
from jax.experimental import pallas as pl
from jax.experimental.pallas import tpu as pltpu
from jax.experimental.pallas import tpu_sc as plsc


_K = 512
_D = 128
_RB = 128
_N_FLAT = 8 * 28 * 28


def _vq_body(flat_ref, cb_ref, idx_ref, oh_ref):
    flat = flat_ref[...]
    cb = cb_ref[...]
    rn = jnp.sum(flat ** 2, axis=1, keepdims=True)
    cn = jnp.sum(cb ** 2, axis=1)
    prod = lax.dot_general(flat, cb, (((1,), (1,)), ((), ())),
                           preferred_element_type=jnp.float32)
    d = rn - 2.0 * prod + cn[None, :]
    dmin = jnp.min(d, axis=1, keepdims=True)
    iota = lax.broadcasted_iota(jnp.int32, d.shape, 1)
    idxv = jnp.min(jnp.where(d == dmin, iota, _K), axis=1)
    oh_ref[...] = (iota == idxv[:, None]).astype(jnp.float32)
    idx_ref[...] = idxv.reshape(1, 1, _RB)


def _vq_tc(flat, codebook):
    nblk = _N_FLAT // _RB
    idx3, onehot = pl.pallas_call(
        _vq_body,
        grid=(nblk,),
        in_specs=[
            pl.BlockSpec((_RB, _D), lambda i: (i, 0)),
            pl.BlockSpec((_K, _D), lambda i: (0, 0)),
        ],
        out_specs=[
            pl.BlockSpec((1, 1, _RB), lambda i: (i, 0, 0)),
            pl.BlockSpec((_RB, _K), lambda i: (i, 0)),
        ],
        out_shape=[
            jax.ShapeDtypeStruct((nblk, 1, _RB), jnp.int32),
            jax.ShapeDtypeStruct((_N_FLAT, _K), jnp.float32),
        ],
    )(flat, codebook)
    return idx3.reshape(_N_FLAT), onehot


_NW = 32
_BPAD = 6400
_BPW = _BPAD // _NW


def _sc_gather(codebook, idx_pad):
    mesh = plsc.VectorSubcoreMesh(core_axis_name="c", subcore_axis_name="s")

    @functools.partial(
        pl.kernel, mesh=mesh,
        out_type=jax.ShapeDtypeStruct((_BPAD, _D), jnp.float32),
        scratch_types=[
            pltpu.VMEM((_BPW,), jnp.int32),
            pltpu.VMEM((_BPW, _D), jnp.float32),
            pltpu.SemaphoreType.DMA,
        ],
    )
    def k(table_hbm, idx_hbm, out_hbm, idx_v, rows_v, sem):
        wid = lax.axis_index("s") * 2 + lax.axis_index("c")
        base = wid * _BPW
        pltpu.sync_copy(idx_hbm.at[pl.ds(base, _BPW)], idx_v)
        pltpu.async_copy(table_hbm.at[idx_v], rows_v, sem).wait()
        pltpu.sync_copy(rows_v, out_hbm.at[pl.ds(base, _BPW)])

    return k(codebook, idx_pad)


def _collapse_w(w4):
    re = jnp.stack([w4[0], w4[1] + w4[2], w4[3]], axis=0)
    ro = jnp.stack([w4[0] + w4[1], w4[2] + w4[3]], axis=0)

    def colc(wr):
        ce = jnp.stack([wr[:, 0], wr[:, 1] + wr[:, 2], wr[:, 3]], axis=1)
        co = jnp.stack([wr[:, 0] + wr[:, 1], wr[:, 2] + wr[:, 3]], axis=1)
        return ce, co

    wee, weo = colc(re)
    woe, woo = colc(ro)
    ci, co = w4.shape[2], w4.shape[3]

    def m(w):
        t, s = w.shape[0], w.shape[1]
        if s == 3:
            w = w[:, jnp.array([1, 2, 0])]
        w = w.reshape(t, s * ci, co)
        if s == 2:
            w = jnp.concatenate([w, jnp.zeros((t, ci, co), jnp.float32)], 1)
        return w

    wee, weo, woe, woo = m(wee), m(weo), m(woe), m(woo)
    z1 = jnp.zeros((1, 3 * ci, co), jnp.float32)
    woe = jnp.concatenate([z1, woe], axis=0)
    woo = jnp.concatenate([z1, woo], axis=0)
    return jnp.concatenate([wee, weo, woe, woo], axis=2)


def _dec_stage_body(Hc, W, Ci, Co, apply_act, rc,
                    x_ref, a_ref, c_ref, wcat_ref, b_ref,
                    ue_ref, uo_ref, st_ref):
    xr = x_ref[0, 0]
    if apply_act:
        xr = jnp.maximum(xr * a_ref[0] + c_ref[0], 0.0)
        k = pl.program_id(1)
        r = lax.broadcasted_iota(jnp.int32, (Hc + 2, 1, 1), 0)
        valid = ((r != 0) | (k != 0)) & ((r != Hc + 1) | (k != rc - 1))
        xr = jnp.where(valid, xr, 0.0)
    zc = jnp.zeros((Hc + 2, 1, Ci), jnp.float32)
    xp = jnp.concatenate([zc, xr, zc], axis=1)
    c3 = jnp.concatenate([xp[:, 1:W + 1], xp[:, 2:W + 2], xp[:, 0:W]],
                         axis=-1)

    def mm(slab, wh, wl):
        s = slab.reshape(Hc * W, 3 * Ci)
        hi = s.astype(jnp.bfloat16)
        lo = (s - hi.astype(jnp.float32)).astype(jnp.bfloat16)
        dn = (((1,), (0,)), ((), ()))
        return (lax.dot_general(hi, wh, dn, preferred_element_type=jnp.float32)
                + (lax.dot_general(hi, wl, dn,
                                   preferred_element_type=jnp.float32)
                   + lax.dot_general(lo, wh, dn,
                                     preferred_element_type=jnp.float32)))

    wh, wl = wcat_ref[0], wcat_ref[1]
    p = mm(c3[0:Hc], wh[0], wl[0]) + mm(c3[1:Hc + 1], wh[1], wl[1]) \
        + mm(c3[2:Hc + 2], wh[2], wl[2]) + b_ref[0]
    ue_ref[0, 0] = p[:, :2 * Co].reshape(Hc, W, 2 * Co)
    uo_ref[0, 0] = p[:, 2 * Co:].reshape(Hc, W, 2 * Co)
    st_ref[0, 0, 0] = jnp.sum(p, axis=0)
    st_ref[0, 0, 1] = jnp.sum(p * p, axis=0)


def _dec_stage(x, a, c, w4, bias, apply_act, rc=1):
    B, H, W, Ci = x.shape
    Co = w4.shape[-1]
    Hc = H // rc
    wcat = _collapse_w(w4)
    wh = wcat.astype(jnp.bfloat16)
    wcat2 = jnp.stack([wh, (wcat - wh.astype(jnp.float32)
                            ).astype(jnp.bfloat16)])
    bias4 = jnp.tile(bias, 4).reshape(1, 4 * Co)
    xpad = jnp.pad(x, ((0, 0), (1, 1), (0, 0), (0, 0)))
    if rc > 1:
        win = jnp.stack([xpad[:, k * Hc:k * Hc + Hc + 2] for k in range(rc)],
                        axis=1)
    else:
        win = xpad[:, None]
    body = functools.partial(_dec_stage_body, Hc, W, Ci, Co, apply_act, rc)
    full = lambda s: pl.BlockSpec(s, lambda b, k: tuple(0 for _ in s))
    outs = pl.pallas_call(
        body,
        grid=(B, rc),
        in_specs=[
            pl.BlockSpec((1, 1, Hc + 2, W, Ci), lambda b, k: (b, k, 0, 0, 0)),
            full((1, Ci)), full((1, Ci)),
            full((2, 3, 3 * Ci, 4 * Co)), full((1, 4 * Co)),
        ],
        out_specs=[
            pl.BlockSpec((1, 1, Hc, W, 2 * Co), lambda b, k: (b, k, 0, 0, 0)),
            pl.BlockSpec((1, 1, Hc, W, 2 * Co), lambda b, k: (b, k, 0, 0, 0)),
            pl.BlockSpec((1, 1, 2, 4 * Co), lambda b, k: (b, k, 0, 0)),
        ],
        out_shape=[
            jax.ShapeDtypeStruct((B, rc, Hc, W, 2 * Co), jnp.float32),
            jax.ShapeDtypeStruct((B, rc, Hc, W, 2 * Co), jnp.float32),
            jax.ShapeDtypeStruct((B, rc, 2, 4 * Co), jnp.float32),
        ],
    )(win, a.reshape(1, Ci), c.reshape(1, Ci), wcat2, bias4)
    ue, uo, st = outs
    ue = ue.reshape(B, H, W, 2 * Co)
    uo = uo.reshape(B, H, W, 2 * Co)
    o = jnp.stack([ue, uo], axis=2)
    st = jnp.sum(st, axis=(0, 1)).reshape(2, 4, Co).sum(axis=1)
    return o.reshape(B, 2 * H, 2 * W, Co), st


def _bn_coeffs(stats, n, gamma, beta, eps=1e-5):
    m = stats[0] / n
    v = stats[1] / n - m * m
    a = gamma / jnp.sqrt(v + eps)
    return a, beta - m * a


def _out_body(x_ref, a_ref, c_ref, w5_ref, bo_ref, rec_ref):
    HO, WO = 55, 220
    x = jnp.maximum(x_ref[0, 0] * a_ref[0] + c_ref[0], 0.0)
    r5 = jnp.concatenate([x[u:u + HO] for u in range(5)], axis=-1)
    s = r5.reshape(HO * 224, 160)
    hi = s.astype(jnp.bfloat16)
    lo = (s - hi.astype(jnp.float32)).astype(jnp.bfloat16)
    dn = (((1,), (0,)), ((), ()))
    g5 = (lax.dot_general(hi, w5_ref[0], dn,
                          preferred_element_type=jnp.float32)
          + (lax.dot_general(hi, w5_ref[1], dn,
                             preferred_element_type=jnp.float32)
             + lax.dot_general(lo, w5_ref[0], dn,
                               preferred_element_type=jnp.float32)))
    g5 = g5.reshape(HO, 224, 8)
    li = lax.broadcasted_iota(jnp.int32, (1, 1, 8), 2)
    acc5 = jnp.zeros((HO, WO, 8), jnp.float32)
    for v in range(5):
        acc5 = acc5 + jnp.where(li == v, g5[:, v:v + WO, :], 0.0)
    rec_ref[0, 0] = jnp.maximum(jnp.sum(acc5, axis=2) + bo_ref[0, 0], 0.0)


def _dec_out(x, a, c, wo, bo):
    B = x.shape[0]
    w5f = jnp.concatenate(
        [jnp.transpose(wo[:, :, :, 0], (0, 2, 1)).reshape(160, 5),
         jnp.zeros((160, 3), jnp.float32)], axis=1)
    w5h = w5f.astype(jnp.bfloat16)
    w5 = jnp.stack([w5h, (w5f - w5h.astype(jnp.float32)
                          ).astype(jnp.bfloat16)])
    win = jnp.stack([x[:, 55 * k:55 * k + 59] for k in range(4)], axis=1)
    rec = pl.pallas_call(
        _out_body,
        grid=(B, 4),
        in_specs=[
            pl.BlockSpec((1, 1, 59, 224, 32), lambda b, k: (b, k, 0, 0, 0)),
            pl.BlockSpec((1, 32), lambda b, k: (0, 0)),
            pl.BlockSpec((1, 32), lambda b, k: (0, 0)),
            pl.BlockSpec((2, 160, 8), lambda b, k: (0, 0, 0)),
            pl.BlockSpec((1, 1), lambda b, k: (0, 0)),
        ],
        out_specs=pl.BlockSpec((1, 1, 55, 220), lambda b, k: (b, k, 0, 0)),
        out_shape=jax.ShapeDtypeStruct((B, 4, 55, 220), jnp.float32),
    )(win, a.reshape(1, 32), c.reshape(1, 32), w5, bo.reshape(1, 1))
    return rec.reshape(B, 220, 220, 1)


def _conv(x, w, b, stride=1, padding='SAME'):
    y = lax.conv_general_dilated(x, w, (stride, stride), padding,
                                 dimension_numbers=('NHWC', 'HWIO', 'NHWC'))
    return y + b


def _bn(x, gamma, beta, eps=1e-5):
    m = jnp.mean(x, axis=(0, 1, 2), keepdims=True)
    v = jnp.var(x, axis=(0, 1, 2), keepdims=True)
    return gamma * (x - m) / jnp.sqrt(v + eps) + beta


def _upsample(x):
    return jnp.repeat(jnp.repeat(x, 2, axis=1), 2, axis=2)


def kernel(img, We1, be1, We2, be2, We3, be3, codebook,
           Wd1, bd1, g1, bb1, Wd2, bd2, g2, bb2, Wd3, bd3, g3, bb3, Wo, bo):
    x = _conv(img, We1, be1, 2)
    x = _conv(x, We2, be2, 2)
    encoded = _conv(x, We3, be3, 2)

    flat = encoded.reshape((-1, _D))
    idx, discrete = _vq_tc(flat, codebook)

    idx_pad = jnp.concatenate(
        [idx, jnp.zeros((_BPAD - _N_FLAT,), jnp.int32)])
    qflat = _sc_gather(codebook, idx_pad)[:_N_FLAT]
    quantized = qflat.reshape(encoded.shape)

    zero128 = jnp.zeros((128,), jnp.float32)
    one128 = jnp.ones((128,), jnp.float32)
    o1, st1 = _dec_stage(quantized, one128, zero128, Wd1, bd1, False)
    a1, c1 = _bn_coeffs(st1, 8 * 56 * 56, g1, bb1)
    o2, st2 = _dec_stage(o1, a1, c1, Wd2, bd2, True)
    a2, c2 = _bn_coeffs(st2, 8 * 112 * 112, g2, bb2)
    o3, st3 = _dec_stage(o2, a2, c2, Wd3, bd3, True, rc=4)
    a3, c3 = _bn_coeffs(st3, 8 * 224 * 224, g3, bb3)
    rec = _dec_out(o3, a3, c3, Wo, bo)
    return (rec, encoded, discrete, quantized)


# speedup vs baseline: 1.7106x; 1.0297x over previous
import functools

import jax
import jax.numpy as jnp
from jax import lax
from jax.experimental import pallas as pl
from jax.experimental.pallas import tpu as pltpu
from jax.experimental.pallas import tpu_sc as plsc


_K = 512
_D = 128
_RB = 128
_N_FLAT = 8 * 28 * 28


def _vq_body(flat_ref, cb_ref, idx_ref, oh_ref):
    flat = flat_ref[...]
    cb = cb_ref[...]
    rn = jnp.sum(flat ** 2, axis=1, keepdims=True)
    cn = jnp.sum(cb ** 2, axis=1)
    prod = lax.dot_general(flat, cb, (((1,), (1,)), ((), ())),
                           preferred_element_type=jnp.float32)
    d = rn - 2.0 * prod + cn[None, :]
    dmin = jnp.min(d, axis=1, keepdims=True)
    iota = lax.broadcasted_iota(jnp.int32, d.shape, 1)
    idxv = jnp.min(jnp.where(d == dmin, iota, _K), axis=1)
    oh_ref[...] = (iota == idxv[:, None]).astype(jnp.float32)
    idx_ref[...] = idxv.reshape(1, 1, _RB)


def _vq_tc(flat, codebook):
    nblk = _N_FLAT // _RB
    idx3, onehot = pl.pallas_call(
        _vq_body,
        grid=(nblk,),
        in_specs=[
            pl.BlockSpec((_RB, _D), lambda i: (i, 0)),
            pl.BlockSpec((_K, _D), lambda i: (0, 0)),
        ],
        out_specs=[
            pl.BlockSpec((1, 1, _RB), lambda i: (i, 0, 0)),
            pl.BlockSpec((_RB, _K), lambda i: (i, 0)),
        ],
        out_shape=[
            jax.ShapeDtypeStruct((nblk, 1, _RB), jnp.int32),
            jax.ShapeDtypeStruct((_N_FLAT, _K), jnp.float32),
        ],
    )(flat, codebook)
    return idx3.reshape(_N_FLAT), onehot


_NW = 32
_BPAD = 6400
_BPW = _BPAD // _NW


def _sc_gather(codebook, idx_pad):
    mesh = plsc.VectorSubcoreMesh(core_axis_name="c", subcore_axis_name="s")

    @functools.partial(
        pl.kernel, mesh=mesh,
        out_type=jax.ShapeDtypeStruct((_BPAD, _D), jnp.float32),
        scratch_types=[
            pltpu.VMEM((_BPW,), jnp.int32),
            pltpu.VMEM((_BPW, _D), jnp.float32),
            pltpu.SemaphoreType.DMA,
        ],
    )
    def k(table_hbm, idx_hbm, out_hbm, idx_v, rows_v, sem):
        wid = lax.axis_index("s") * 2 + lax.axis_index("c")
        base = wid * _BPW
        pltpu.sync_copy(idx_hbm.at[pl.ds(base, _BPW)], idx_v)
        pltpu.async_copy(table_hbm.at[idx_v], rows_v, sem).wait()
        pltpu.sync_copy(rows_v, out_hbm.at[pl.ds(base, _BPW)])

    return k(codebook, idx_pad)


def _collapse_w(w4):
    re = jnp.stack([w4[0], w4[1] + w4[2], w4[3]], axis=0)
    ro = jnp.stack([w4[0] + w4[1], w4[2] + w4[3]], axis=0)

    def colc(wr):
        ce = jnp.stack([wr[:, 0], wr[:, 1] + wr[:, 2], wr[:, 3]], axis=1)
        co = jnp.stack([wr[:, 0] + wr[:, 1], wr[:, 2] + wr[:, 3]], axis=1)
        return ce, co

    wee, weo = colc(re)
    woe, woo = colc(ro)
    ci, co = w4.shape[2], w4.shape[3]

    def m(w):
        t, s = w.shape[0], w.shape[1]
        if s == 3:
            w = w[:, jnp.array([1, 2, 0])]
        w = w.reshape(t, s * ci, co)
        if s == 2:
            w = jnp.concatenate([w, jnp.zeros((t, ci, co), jnp.float32)], 1)
        return w

    wee, weo, woe, woo = m(wee), m(weo), m(woe), m(woo)
    z1 = jnp.zeros((1, 3 * ci, co), jnp.float32)
    woe = jnp.concatenate([z1, woe], axis=0)
    woo = jnp.concatenate([z1, woo], axis=0)
    return jnp.concatenate([wee, weo, woe, woo], axis=2)


def _dec_stage_body(Hc, W, Ci, Co, apply_act, rc,
                    x_ref, a_ref, c_ref, wcat_ref, b_ref,
                    ue_ref, uo_ref, st_ref):
    xr = x_ref[0, 0]
    if apply_act:
        xr = jnp.maximum(xr * a_ref[0] + c_ref[0], 0.0)
        k = pl.program_id(1)
        r = lax.broadcasted_iota(jnp.int32, (Hc + 2, 1, 1), 0)
        valid = ((r != 0) | (k != 0)) & ((r != Hc + 1) | (k != rc - 1))
        xr = jnp.where(valid, xr, 0.0)
    zc = jnp.zeros((Hc + 2, 1, Ci), jnp.float32)
    xp = jnp.concatenate([zc, xr, zc], axis=1)
    c3 = jnp.concatenate([xp[:, 1:W + 1], xp[:, 2:W + 2], xp[:, 0:W]],
                         axis=-1)

    def mm(slab, wh, wl):
        s = slab.reshape(Hc * W, 3 * Ci)
        hi = s.astype(jnp.bfloat16)
        lo = (s - hi.astype(jnp.float32)).astype(jnp.bfloat16)
        dn = (((1,), (0,)), ((), ()))
        return (lax.dot_general(hi, wh, dn, preferred_element_type=jnp.float32)
                + (lax.dot_general(hi, wl, dn,
                                   preferred_element_type=jnp.float32)
                   + lax.dot_general(lo, wh, dn,
                                     preferred_element_type=jnp.float32)))

    wh, wl = wcat_ref[0], wcat_ref[1]
    p = mm(c3[0:Hc], wh[0], wl[0]) + mm(c3[1:Hc + 1], wh[1], wl[1]) \
        + mm(c3[2:Hc + 2], wh[2], wl[2]) + b_ref[0]
    ue_ref[0, 0] = p[:, :2 * Co].reshape(Hc, W, 2 * Co)
    uo_ref[0, 0] = p[:, 2 * Co:].reshape(Hc, W, 2 * Co)
    st_ref[0, 0, 0] = jnp.sum(p, axis=0)
    st_ref[0, 0, 1] = jnp.sum(p * p, axis=0)


def _dec_stage(x, a, c, w4, bias, apply_act, rc=1):
    B, H, W, Ci = x.shape
    Co = w4.shape[-1]
    Hc = H // rc
    wcat = _collapse_w(w4)
    wh = wcat.astype(jnp.bfloat16)
    wcat2 = jnp.stack([wh, (wcat - wh.astype(jnp.float32)
                            ).astype(jnp.bfloat16)])
    bias4 = jnp.tile(bias, 4).reshape(1, 4 * Co)
    xpad = jnp.pad(x, ((0, 0), (1, 1), (0, 0), (0, 0)))
    if rc > 1:
        win = jnp.stack([xpad[:, k * Hc:k * Hc + Hc + 2] for k in range(rc)],
                        axis=1)
    else:
        win = xpad[:, None]
    body = functools.partial(_dec_stage_body, Hc, W, Ci, Co, apply_act, rc)
    full = lambda s: pl.BlockSpec(s, lambda b, k: tuple(0 for _ in s))
    outs = pl.pallas_call(
        body,
        grid=(B, rc),
        in_specs=[
            pl.BlockSpec((1, 1, Hc + 2, W, Ci), lambda b, k: (b, k, 0, 0, 0)),
            full((1, Ci)), full((1, Ci)),
            full((2, 3, 3 * Ci, 4 * Co)), full((1, 4 * Co)),
        ],
        out_specs=[
            pl.BlockSpec((1, 1, Hc, W, 2 * Co), lambda b, k: (b, k, 0, 0, 0)),
            pl.BlockSpec((1, 1, Hc, W, 2 * Co), lambda b, k: (b, k, 0, 0, 0)),
            pl.BlockSpec((1, 1, 2, 4 * Co), lambda b, k: (b, k, 0, 0)),
        ],
        out_shape=[
            jax.ShapeDtypeStruct((B, rc, Hc, W, 2 * Co), jnp.float32),
            jax.ShapeDtypeStruct((B, rc, Hc, W, 2 * Co), jnp.float32),
            jax.ShapeDtypeStruct((B, rc, 2, 4 * Co), jnp.float32),
        ],
    )(win, a.reshape(1, Ci), c.reshape(1, Ci), wcat2, bias4)
    ue, uo, st = outs
    ue = ue.reshape(B, H, W, 2 * Co)
    uo = uo.reshape(B, H, W, 2 * Co)
    o = jnp.stack([ue, uo], axis=2)
    st = jnp.sum(st, axis=(0, 1)).reshape(2, 4, Co).sum(axis=1)
    return o.reshape(B, 2 * H, 2 * W, Co), st


def _bn_coeffs(stats, n, gamma, beta, eps=1e-5):
    m = stats[0] / n
    v = stats[1] / n - m * m
    a = gamma / jnp.sqrt(v + eps)
    return a, beta - m * a


def _out_body(x_ref, a_ref, c_ref, w5_ref, bo_ref, rec_ref):
    HO, WO = 55, 220
    x = jnp.maximum(x_ref[0, 0] * a_ref[0] + c_ref[0], 0.0)
    r5 = jnp.concatenate([x[u:u + HO] for u in range(5)], axis=-1)
    hi = r5.reshape(HO * 224, 160).astype(jnp.bfloat16)
    g5 = lax.dot_general(hi, w5_ref[0], (((1,), (0,)), ((), ())),
                         preferred_element_type=jnp.float32)
    g5 = g5.reshape(HO, 224, 8)
    li = lax.broadcasted_iota(jnp.int32, (1, 1, 8), 2)
    acc5 = jnp.zeros((HO, WO, 8), jnp.float32)
    for v in range(5):
        acc5 = acc5 + jnp.where(li == v, g5[:, v:v + WO, :], 0.0)
    rec_ref[0, 0] = jnp.maximum(jnp.sum(acc5, axis=2) + bo_ref[0, 0], 0.0)


def _dec_out(x, a, c, wo, bo):
    B = x.shape[0]
    w5f = jnp.concatenate(
        [jnp.transpose(wo[:, :, :, 0], (0, 2, 1)).reshape(160, 5),
         jnp.zeros((160, 3), jnp.float32)], axis=1)
    w5h = w5f.astype(jnp.bfloat16)
    w5 = jnp.stack([w5h, (w5f - w5h.astype(jnp.float32)
                          ).astype(jnp.bfloat16)])
    win = jnp.stack([x[:, 55 * k:55 * k + 59] for k in range(4)], axis=1)
    rec = pl.pallas_call(
        _out_body,
        grid=(B, 4),
        in_specs=[
            pl.BlockSpec((1, 1, 59, 224, 32), lambda b, k: (b, k, 0, 0, 0)),
            pl.BlockSpec((1, 32), lambda b, k: (0, 0)),
            pl.BlockSpec((1, 32), lambda b, k: (0, 0)),
            pl.BlockSpec((2, 160, 8), lambda b, k: (0, 0, 0)),
            pl.BlockSpec((1, 1), lambda b, k: (0, 0)),
        ],
        out_specs=pl.BlockSpec((1, 1, 55, 220), lambda b, k: (b, k, 0, 0)),
        out_shape=jax.ShapeDtypeStruct((B, 4, 55, 220), jnp.float32),
    )(win, a.reshape(1, 32), c.reshape(1, 32), w5, bo.reshape(1, 1))
    return rec.reshape(B, 220, 220, 1)


def _conv(x, w, b, stride=1, padding='SAME'):
    y = lax.conv_general_dilated(x, w, (stride, stride), padding,
                                 dimension_numbers=('NHWC', 'HWIO', 'NHWC'))
    return y + b


def _bn(x, gamma, beta, eps=1e-5):
    m = jnp.mean(x, axis=(0, 1, 2), keepdims=True)
    v = jnp.var(x, axis=(0, 1, 2), keepdims=True)
    return gamma * (x - m) / jnp.sqrt(v + eps) + beta


def _upsample(x):
    return jnp.repeat(jnp.repeat(x, 2, axis=1), 2, axis=2)


def kernel(img, We1, be1, We2, be2, We3, be3, codebook,
           Wd1, bd1, g1, bb1, Wd2, bd2, g2, bb2, Wd3, bd3, g3, bb3, Wo, bo):
    x = _conv(img, We1, be1, 2)
    x = _conv(x, We2, be2, 2)
    encoded = _conv(x, We3, be3, 2)

    flat = encoded.reshape((-1, _D))
    idx, discrete = _vq_tc(flat, codebook)

    idx_pad = jnp.concatenate(
        [idx, jnp.zeros((_BPAD - _N_FLAT,), jnp.int32)])
    qflat = _sc_gather(codebook, idx_pad)[:_N_FLAT]
    quantized = qflat.reshape(encoded.shape)

    zero128 = jnp.zeros((128,), jnp.float32)
    one128 = jnp.ones((128,), jnp.float32)
    o1, st1 = _dec_stage(quantized, one128, zero128, Wd1, bd1, False)
    a1, c1 = _bn_coeffs(st1, 8 * 56 * 56, g1, bb1)
    o2, st2 = _dec_stage(o1, a1, c1, Wd2, bd2, True)
    a2, c2 = _bn_coeffs(st2, 8 * 112 * 112, g2, bb2)
    o3, st3 = _dec_stage(o2, a2, c2, Wd3, bd3, True, rc=4)
    a3, c3 = _bn_coeffs(st3, 8 * 224 * 224, g3, bb3)
    rec = _dec_out(o3, a3, c3, Wo, bo)
    return (rec, encoded, discrete, quantized)
